# baseline (speedup 1.0000x reference)
import functools

import jax
import jax.numpy as jnp
from jax import lax
from jax.experimental import pallas as pl
from jax.experimental.pallas import tpu as pltpu
from jax.experimental.pallas import tpu_sc as plsc

D_MODEL = 768
NUM_ROWS = 6
BATCH = 16384

_info = plsc.get_sparse_core_info()
NUM_CORES = _info.num_cores
NUM_SUBCORES = _info.num_subcores
NUM_WORKERS = NUM_CORES * NUM_SUBCORES
B_PER_W = BATCH // NUM_WORKERS
LANES = 16
N_GROUPS = B_PER_W // LANES

_mesh = plsc.VectorSubcoreMesh(core_axis_name="c", subcore_axis_name="s")


@functools.partial(
    pl.kernel,
    mesh=_mesh,
    out_type=jax.ShapeDtypeStruct((BATCH, D_MODEL), jnp.float32),
    scratch_types=[
        pltpu.VMEM((B_PER_W,), jnp.int32),
        pltpu.VMEM((NUM_ROWS, D_MODEL), jnp.float32),
        pltpu.SemaphoreType.DMA,
    ],
)
def _emb_kernel(idx_hbm, table_hbm, out_hbm, idx_v, table_v, wsem):
    cid = lax.axis_index("c")
    sid = lax.axis_index("s")
    wid = sid * NUM_CORES + cid
    base = wid * B_PER_W

    pltpu.sync_copy(table_hbm, table_v)
    pltpu.sync_copy(idx_hbm.at[wid], idx_v)

    def group_body(g, _):
        v = idx_v[pl.ds(g * LANES, LANES)]
        b = base + g * LANES
        for l in range(LANES):
            e = v[l]
            pltpu.make_async_copy(
                table_v.at[e], out_hbm.at[b + l], wsem
            ).start()
        return 0

    lax.fori_loop(0, N_GROUPS, group_body, 0)

    pltpu.make_async_copy(
        out_hbm.at[pl.ds(base, B_PER_W)],
        out_hbm.at[pl.ds(base, B_PER_W)],
        wsem,
    ).wait()


def kernel(emotion_id, table):
    if emotion_id.ndim > 1:
        emotion_id = emotion_id.reshape(-1)
    idx = emotion_id.astype(jnp.int32).reshape(NUM_WORKERS, B_PER_W)
    return _emb_kernel(idx, table)
